# Initial kernel scaffold; baseline (speedup 1.0000x reference)
import jax
import jax.numpy as jnp
from jax.experimental import pallas as pl


def kernel(x, edge_index, edge_weight, W_in, b_in, delta, a, W_out, b_out):
    raise NotImplementedError("write your pallas kernel here")


# speedup vs baseline: 4.1160x; 4.1160x over previous
import functools

import jax
import jax.numpy as jnp
from jax import lax
from jax.experimental import pallas as pl
from jax.experimental.pallas import tpu as pltpu
from jax.experimental.pallas import tpu_sc as plsc

N = 10000
E = 320000
D = 128
NCLS = 2

NCORE = 2
NSUB = 16
LANES = 16

NH = N // NCORE
TRASH = NH
ACC_ROWS = 5120
CH = 128
NCH = 157
EPAD = NSUB * NCH * CH - E

_MESH = plsc.VectorSubcoreMesh(core_axis_name="c", subcore_axis_name="s")


def _spmm_body(table, srcr, dstr, wr, out, src_v, dst_v, w_v, rows, zblk,
               acc, sem):
    c = lax.axis_index("c")
    s = lax.axis_index("s")

    pltpu.sync_copy(srcr.at[s], src_v)
    pltpu.sync_copy(dstr.at[c, s], dst_v)
    pltpu.sync_copy(wr.at[s], w_v)

    for i in range(16):
        for g in range(8):
            zblk[i, pl.ds(g * 16, 16)] = jnp.zeros((16,), jnp.float32)

    def zbody(k, _):
        pltpu.sync_copy(zblk, acc.at[pl.ds(s * 320 + k * 16, 16)])
        return 0

    lax.fori_loop(0, 20, zbody, 0)
    plsc.subcore_barrier()

    def chunk(j, _):
        pltpu.async_copy(table.at[src_v.at[j]], rows, sem).wait()

        def scale(g, _):
            w16 = w_v[j, pl.ds(g * 16, 16)]
            base = g * 16
            for el in range(16):
                e = base + el
                wsc = w16[el]
                for gg in range(8):
                    sl = pl.ds(gg * 16, 16)
                    rows[e, sl] = rows[e, sl] * wsc
            return 0

        lax.fori_loop(0, CH // 16, scale, 0)
        pltpu.sync_copy(rows, acc.at[dst_v.at[j]], add=True)
        return 0

    lax.fori_loop(0, NCH, chunk, 0)
    plsc.subcore_barrier()

    @pl.when(s < 15)
    def _():
        off = s * 312
        pltpu.sync_copy(acc.at[pl.ds(off, 312)],
                        out.at[pl.ds(c * NH + off, 312)])

    @pl.when(s == 15)
    def _():
        pltpu.sync_copy(acc.at[pl.ds(4680, 320)],
                        out.at[pl.ds(c * NH + 4680, 320)])


_spmm = functools.partial(
    pl.kernel,
    out_type=jax.ShapeDtypeStruct((N, D), jnp.float32),
    mesh=_MESH,
    scratch_types=[
        pltpu.VMEM((NCH, CH), jnp.int32),
        pltpu.VMEM((NCH, CH), jnp.int32),
        pltpu.VMEM((NCH, CH), jnp.float32),
        pltpu.VMEM((CH, D), jnp.float32),
        pltpu.VMEM((16, D), jnp.float32),
        pltpu.VMEM_SHARED((ACC_ROWS, D), jnp.float32),
        pltpu.SemaphoreType.DMA,
    ],
)(_spmm_body)


def _spmm1_body(table, srcr, dstr, wr, out, tab_v, src_v, dst_v, w_v,
                prod_v, zv, tmp_v, acc, sem):
    c = lax.axis_index("c")
    s = lax.axis_index("s")

    pltpu.sync_copy(table, tab_v)
    pltpu.sync_copy(srcr.at[s], src_v)
    pltpu.sync_copy(dstr.at[c, s], dst_v)
    pltpu.sync_copy(wr.at[s], w_v)

    for g in range(10):
        zv[pl.ds(g * 16, 16)] = jnp.zeros((16,), jnp.float32)
    pltpu.sync_copy(zv, acc.at[pl.ds(s * 320, 160)])
    pltpu.sync_copy(zv, acc.at[pl.ds(s * 320 + 160, 160)])
    plsc.subcore_barrier()

    def chunk(j, _):
        for g in range(8):
            sl = pl.ds(g * 16, 16)
            vals = plsc.load_gather(tab_v, [src_v[j, sl]])
            prod_v[sl] = vals * w_v[j, sl]
        pltpu.sync_copy(prod_v, acc.at[dst_v.at[j]], add=True)
        return 0

    lax.fori_loop(0, NCH, chunk, 0)
    plsc.subcore_barrier()

    pltpu.sync_copy(acc.at[pl.ds(s * 320, 320)], tmp_v)
    pltpu.sync_copy(tmp_v, out.at[pl.ds(c * ACC_ROWS + s * 320, 320)])


_spmm1 = functools.partial(
    pl.kernel,
    out_type=jax.ShapeDtypeStruct((2 * ACC_ROWS,), jnp.float32),
    mesh=_MESH,
    compiler_params=pltpu.CompilerParams(needs_layout_passes=False),
    scratch_types=[
        pltpu.VMEM((N,), jnp.float32),
        pltpu.VMEM((NCH, CH), jnp.int32),
        pltpu.VMEM((NCH, CH), jnp.int32),
        pltpu.VMEM((NCH, CH), jnp.float32),
        pltpu.VMEM((CH,), jnp.float32),
        pltpu.VMEM((160,), jnp.float32),
        pltpu.VMEM((320,), jnp.float32),
        pltpu.VMEM_SHARED((ACC_ROWS,), jnp.float32),
        pltpu.SemaphoreType.DMA,
    ],
)(_spmm1_body)


def _dense_in_body(xr, wr, br, hr):
    hr[...] = jnp.tanh(
        jnp.dot(xr[...], wr[...], preferred_element_type=jnp.float32)
        + br[...])


def _combine_body(hr, t1, t2, t3, cr, lowr, highr, dnr):
    h = hr[...]
    x1 = t1[...]
    x2 = t2[...]
    x3 = t3[...]
    low = x3 + cr[0, 0] * x2 + cr[0, 1] * x1 + cr[0, 2] * h
    high = x3 + cr[0, 3] * x2 + cr[0, 4] * x1 + cr[0, 5] * h
    lowr[...] = low
    highr[...] = high
    dv = low - h
    dnr[...] = jnp.sqrt(jnp.sum(dv * dv, axis=1, keepdims=True))


def _minmax_body(hdr, outr):
    hd = hdr[...]
    mn = jnp.min(hd)
    mx = jnp.max(hd)
    outr[...] = jnp.concatenate(
        [jnp.full((1, 64), mn, jnp.float32),
         jnp.full((1, 64), mx, jnp.float32)], axis=1)


def _final_body(lowr, highr, hdr, mmr, wr, br, outr):
    mn = mmr[0, 0]
    mx = mmr[0, 64]
    nd = (hdr[...] - mn) / (mx - mn)
    fin = (1.0 - nd) * lowr[...] + nd * highr[...]
    y = jnp.dot(jnp.maximum(fin, 0.0), wr[...],
                preferred_element_type=jnp.float32) + br[...]
    m = jnp.max(y, axis=1, keepdims=True)
    sh = y - m
    outr[...] = sh - jnp.log(jnp.sum(jnp.exp(sh), axis=1, keepdims=True))


def kernel(x, edge_index, edge_weight, W_in, b_in, delta, a, W_out, b_out):
    dst = edge_index[0]
    src = edge_index[1]

    src_p = jnp.concatenate(
        [src, jnp.zeros((EPAD,), jnp.int32)]).reshape(NSUB, NCH, CH)
    w_p = jnp.concatenate(
        [edge_weight, jnp.zeros((EPAD,), jnp.float32)]).reshape(NSUB, NCH, CH)
    dst_cores = []
    for c in range(NCORE):
        base = c * NH
        in_range = (dst >= base) & (dst < base + NH)
        adj = jnp.where(in_range, dst - base, TRASH)
        dst_cores.append(
            jnp.concatenate([adj, jnp.full((EPAD,), TRASH, jnp.int32)]))
    dst_p = jnp.stack(dst_cores).reshape(NCORE, NSUB, NCH, CH)

    h = pl.pallas_call(
        _dense_in_body,
        grid=(10,),
        in_specs=[
            pl.BlockSpec((1000, D), lambda i: (i, 0)),
            pl.BlockSpec((D, D), lambda i: (0, 0)),
            pl.BlockSpec((1, D), lambda i: (0, 0)),
        ],
        out_specs=pl.BlockSpec((1000, D), lambda i: (i, 0)),
        out_shape=jax.ShapeDtypeStruct((N, D), jnp.float32),
    )(x, W_in, b_in.reshape(1, D))

    tx1 = _spmm(h, src_p, dst_p, w_p)
    tx2 = _spmm(tx1, src_p, dst_p, w_p)
    tx3 = _spmm(tx2, src_p, dst_p, w_p)

    d = delta[0]
    av = a[0]
    coef = jnp.stack([
        -3.0 * d - av,
        3.0 * d ** 2 + 2.0 * d * av,
        -(d ** 3 + d ** 2 * av),
        -3.0 * d + av,
        3.0 * d ** 2 - 2.0 * d * av,
        d ** 2 * av - d ** 3,
    ]).reshape(1, 6)

    low, high, dn = pl.pallas_call(
        _combine_body,
        grid=(10,),
        in_specs=[
            pl.BlockSpec((1000, D), lambda i: (i, 0)),
            pl.BlockSpec((1000, D), lambda i: (i, 0)),
            pl.BlockSpec((1000, D), lambda i: (i, 0)),
            pl.BlockSpec((1000, D), lambda i: (i, 0)),
            pl.BlockSpec(memory_space=pltpu.SMEM),
        ],
        out_specs=[
            pl.BlockSpec((1000, D), lambda i: (i, 0)),
            pl.BlockSpec((1000, D), lambda i: (i, 0)),
            pl.BlockSpec((1000, 1), lambda i: (i, 0)),
        ],
        out_shape=[
            jax.ShapeDtypeStruct((N, D), jnp.float32),
            jax.ShapeDtypeStruct((N, D), jnp.float32),
            jax.ShapeDtypeStruct((N, 1), jnp.float32),
        ],
    )(h, tx1, tx2, tx3, coef)

    hdp = _spmm1(dn.reshape(N), src_p, dst_p, w_p)
    hd2 = jnp.concatenate(
        [hdp[:NH], hdp[ACC_ROWS:ACC_ROWS + NH]]).reshape(N, 1)

    mm = pl.pallas_call(
        _minmax_body,
        out_shape=jax.ShapeDtypeStruct((1, 128), jnp.float32),
    )(hd2)

    W_pad = jnp.concatenate(
        [W_out, jnp.zeros((D, 128 - NCLS), jnp.float32)], axis=1)
    b_pad = jnp.concatenate(
        [b_out, jnp.full((128 - NCLS,), -1e30, jnp.float32)]).reshape(1, 128)

    out = pl.pallas_call(
        _final_body,
        grid=(10,),
        in_specs=[
            pl.BlockSpec((1000, D), lambda i: (i, 0)),
            pl.BlockSpec((1000, D), lambda i: (i, 0)),
            pl.BlockSpec((1000, 1), lambda i: (i, 0)),
            pl.BlockSpec(memory_space=pltpu.SMEM),
            pl.BlockSpec((D, 128), lambda i: (0, 0)),
            pl.BlockSpec((1, 128), lambda i: (0, 0)),
        ],
        out_specs=pl.BlockSpec((1000, 128), lambda i: (i, 0)),
        out_shape=jax.ShapeDtypeStruct((N, 128), jnp.float32),
    )(low, high, hd2, mm, W_pad, b_pad)

    return out[:, :NCLS]
